# Initial kernel scaffold; baseline (speedup 1.0000x reference)
import jax
import jax.numpy as jnp
from jax.experimental import pallas as pl


def kernel(feat, edge_index, theta_w, theta_b, phi_w, phi_b):
    raise NotImplementedError("write your pallas kernel here")


# speedup vs baseline: 2.1385x; 2.1385x over previous
import functools
import jax
import jax.numpy as jnp
from jax import lax
from jax.experimental import pallas as pl
from jax.experimental.pallas import tpu as pltpu
from jax.experimental.pallas import tpu_sc as plsc

N_NODES = 10000
N_EDGES = 320000
D = 128
L = 16
NF = D // L

NC, NS = 2, 16
NW = NC * NS
R = 320
N_PAD = NW * R
R_ACC = R + 1
DUMP = R
SENT = DUMP << 14

C = 3200
NCHUNK = N_EDGES // C
G = 16

M_PAD = 10240
TC_BLK = 512


def _tc_body(x_ref, wt_ref, wd_ref, bt_ref, bd_ref, h_ref, d_ref):
    x = x_ref[...]
    h_ref[...] = jnp.dot(x, wt_ref[...], preferred_element_type=jnp.float32) + bt_ref[0:1, :]
    d_ref[...] = jnp.dot(x, wd_ref[...], preferred_element_type=jnp.float32) + bd_ref[0:1, :]


def _tc_matmuls(x_pad, wt, wd, bt, bd):
    grid = (M_PAD // TC_BLK,)
    return pl.pallas_call(
        _tc_body,
        grid=grid,
        in_specs=[
            pl.BlockSpec((TC_BLK, D), lambda i: (i, 0)),
            pl.BlockSpec((D, D), lambda i: (0, 0)),
            pl.BlockSpec((D, D), lambda i: (0, 0)),
            pl.BlockSpec((8, D), lambda i: (0, 0)),
            pl.BlockSpec((8, D), lambda i: (0, 0)),
        ],
        out_specs=[
            pl.BlockSpec((TC_BLK, D), lambda i: (i, 0)),
            pl.BlockSpec((TC_BLK, D), lambda i: (i, 0)),
        ],
        out_shape=[
            jax.ShapeDtypeStruct((M_PAD, D), jnp.float32),
            jax.ShapeDtypeStruct((M_PAD, D), jnp.float32),
        ],
    )(x_pad, wt, wd, bt, bd)


def _sc_body(h_hbm, dif_hbm, src_hbm, dst_hbm, out_hbm,
             acc, srcbuf, dstbuf, mkey, gidx, rows, difbuf, sem):
    wid = lax.axis_index("s") * NC + lax.axis_index("c")
    node_base = wid * R

    neg_inf = jnp.full((L,), -jnp.inf, jnp.float32)

    @pl.loop(0, R_ACC)
    def _(r):
        for f in range(NF):
            acc[r, pl.ds(f * L, L)] = neg_inf

    @pl.loop(0, C // L)
    def _(i):
        mkey[pl.ds(i * L, L)] = jnp.full((L,), SENT, jnp.int32)

    @pl.loop(0, NCHUNK)
    def _(c):
        ebase = c * C
        pltpu.sync_copy(dst_hbm.at[pl.ds(ebase, C)], dstbuf)
        pltpu.sync_copy(src_hbm.at[pl.ds(ebase, C)], srcbuf)

        def scan_body(i, cnt):
            d = dstbuf[pl.ds(i * L, L)]
            s = srcbuf[pl.ds(i * L, L)]
            ld = d - node_base
            m = (ld >= 0) & (ld < R)
            key = (ld << 14) | s
            zero = jnp.zeros((L,), jnp.int32)
            mi = jnp.where(m, jnp.ones((L,), jnp.int32), zero)
            pc = plsc.cumsum(mi)
            offs = cnt + pc - 1
            plsc.store_scatter(mkey, [offs], key, mask=m)
            return cnt + pc[L - 1]

        nm = lax.fori_loop(0, C // L, scan_body, jnp.int32(0))
        ng = (nm + (G - 1)) >> 4

        @pl.loop(0, ng)
        def _(g):
            kv = mkey[pl.ds(g * G, L)]
            gidx[pl.ds(0, L)] = kv & 0x3FFF
            pltpu.async_copy(h_hbm.at[gidx], rows, sem).wait()
            ldv = jax.lax.shift_right_logical(kv, 14)
            for j in range(G):
                ld = ldv[j]
                for f in range(NF):
                    sl = pl.ds(f * L, L)
                    acc[ld, sl] = jnp.maximum(acc[ld, sl], rows[j, sl])

    pltpu.sync_copy(dif_hbm.at[pl.ds(node_base, R)], difbuf)

    @pl.loop(0, R)
    def _(r):
        for f in range(NF):
            sl = pl.ds(f * L, L)
            a = acc[r, sl]
            acc[r, sl] = jnp.where(a == neg_inf, jnp.zeros((L,), jnp.float32),
                                   a + difbuf[r, sl])

    pltpu.sync_copy(acc.at[pl.ds(0, R)], out_hbm.at[pl.ds(node_base, R)])


def _sc_segment_max(h_pad, dif_pad, src, dst):
    mesh = plsc.VectorSubcoreMesh(core_axis_name="c", subcore_axis_name="s")
    return pl.kernel(
        _sc_body,
        out_type=jax.ShapeDtypeStruct((N_PAD, D), jnp.float32),
        mesh=mesh,
        compiler_params=pltpu.CompilerParams(needs_layout_passes=False),
        scratch_types=[
            pltpu.VMEM((R_ACC, D), jnp.float32),
            pltpu.VMEM((C,), jnp.int32),
            pltpu.VMEM((C,), jnp.int32),
            pltpu.VMEM((C,), jnp.int32),
            pltpu.VMEM((L,), jnp.int32),
            pltpu.VMEM((G, D), jnp.float32),
            pltpu.VMEM((R, D), jnp.float32),
            pltpu.SemaphoreType.DMA,
        ],
    )(h_pad, dif_pad, src, dst)


@jax.jit
def kernel(feat, edge_index, theta_w, theta_b, phi_w, phi_b):
    x_pad = jnp.zeros((M_PAD, D), jnp.float32).at[:N_NODES].set(feat)
    wt = theta_w.T
    wd = (phi_w - theta_w).T
    bt = jnp.tile(theta_b[None, :], (8, 1))
    bd = jnp.tile((phi_b - theta_b)[None, :], (8, 1))
    h_pad, dif_pad = _tc_matmuls(x_pad, wt, wd, bt, bd)

    src = edge_index[0].astype(jnp.int32)
    dst = edge_index[1].astype(jnp.int32)
    out_pad = _sc_segment_max(h_pad, dif_pad, src, dst)
    return out_pad[:N_NODES]


# speedup vs baseline: 2.4008x; 1.1226x over previous
import functools
import jax
import jax.numpy as jnp
from jax import lax
from jax.experimental import pallas as pl
from jax.experimental.pallas import tpu as pltpu
from jax.experimental.pallas import tpu_sc as plsc

N_NODES = 10000
N_EDGES = 320000
D = 128
L = 16
NF = D // L

NC, NS = 2, 16
NW = NC * NS
R = 320
N_PAD = NW * R
R_ACC = R + 1
DUMP = R
SENT = DUMP << 14

C = 3200
NCHUNK = N_EDGES // C
G = 32
GV = G // L

M_PAD = 10240
TC_BLK = 512


def _tc_body(x_ref, wt_ref, wd_ref, bt_ref, bd_ref, h_ref, d_ref):
    x = x_ref[...]
    h_ref[...] = jnp.dot(x, wt_ref[...], preferred_element_type=jnp.float32) + bt_ref[0:1, :]
    d_ref[...] = jnp.dot(x, wd_ref[...], preferred_element_type=jnp.float32) + bd_ref[0:1, :]


def _tc_matmuls(x_pad, wt, wd, bt, bd):
    grid = (M_PAD // TC_BLK,)
    return pl.pallas_call(
        _tc_body,
        grid=grid,
        in_specs=[
            pl.BlockSpec((TC_BLK, D), lambda i: (i, 0)),
            pl.BlockSpec((D, D), lambda i: (0, 0)),
            pl.BlockSpec((D, D), lambda i: (0, 0)),
            pl.BlockSpec((8, D), lambda i: (0, 0)),
            pl.BlockSpec((8, D), lambda i: (0, 0)),
        ],
        out_specs=[
            pl.BlockSpec((TC_BLK, D), lambda i: (i, 0)),
            pl.BlockSpec((TC_BLK, D), lambda i: (i, 0)),
        ],
        out_shape=[
            jax.ShapeDtypeStruct((M_PAD, D), jnp.float32),
            jax.ShapeDtypeStruct((M_PAD, D), jnp.float32),
        ],
    )(x_pad, wt, wd, bt, bd)


def _sc_body(h_hbm, dif_hbm, src_hbm, dst_hbm, out_hbm,
             acc, srcbuf, dstbuf, mkey, gidx, rows, difbuf,
             esem0, esem1, gsem0, gsem1):
    wid = lax.axis_index("s") * NC + lax.axis_index("c")
    node_base = wid * R

    neg_inf = jnp.full((L,), -jnp.inf, jnp.float32)
    esems = (esem0, esem1)
    gsems = (gsem0, gsem1)

    @pl.loop(0, R_ACC)
    def _(r):
        for f in range(NF):
            acc[r, pl.ds(f * L, L)] = neg_inf

    @pl.loop(0, C // L)
    def _(i):
        mkey[pl.ds(i * L, L)] = jnp.full((L,), SENT, jnp.int32)

    def fire_chunk(c, b):
        eb = c * C
        pltpu.async_copy(dst_hbm.at[pl.ds(eb, C)], dstbuf.at[b], esems[b])
        pltpu.async_copy(src_hbm.at[pl.ds(eb, C)], srcbuf.at[b], esems[b])

    def wait_chunk(b):
        pltpu.make_async_copy(dst_hbm.at[pl.ds(0, C)], dstbuf.at[b], esems[b]).wait()
        pltpu.make_async_copy(src_hbm.at[pl.ds(0, C)], srcbuf.at[b], esems[b]).wait()

    def fire_group(g, b):
        for k in range(GV):
            kv = mkey[pl.ds(g * G + k * L, L)]
            gidx[b, pl.ds(k * L, L)] = kv & 0x3FFF
        pltpu.async_copy(h_hbm.at[gidx.at[b]], rows.at[b], gsems[b])

    def wait_group(b):
        pltpu.make_async_copy(h_hbm.at[gidx.at[b]], rows.at[b], gsems[b]).wait()

    def process_group(g, b):
        for k in range(GV):
            kv = mkey[pl.ds(g * G + k * L, L)]
            ldv = jax.lax.shift_right_logical(kv, 14)
            for j in range(L):
                ld = ldv[j]
                jj = k * L + j
                for f in range(NF):
                    sl = pl.ds(f * L, L)
                    acc[ld, sl] = jnp.maximum(acc[ld, sl], rows[b, jj, sl])

    fire_chunk(0, 0)

    @pl.loop(0, NCHUNK // 2)
    def _(p):
        for b in (0, 1):
            c = 2 * p + b
            wait_chunk(b)

            @pl.when(c + 1 < NCHUNK)
            def _():
                fire_chunk(c + 1, 1 - b)

            def scan_body(i, cnt_v):
                d = dstbuf[b, pl.ds(i * L, L)]
                s = srcbuf[b, pl.ds(i * L, L)]
                ld = d - node_base
                m = (ld >= 0) & (ld < R)
                key = (ld << 14) | s
                zero = jnp.zeros((L,), jnp.int32)
                mi = jnp.where(m, jnp.ones((L,), jnp.int32), zero)
                pc = plsc.cumsum(mi)
                offs = cnt_v + pc - 1
                plsc.store_scatter(mkey, [offs], key, mask=m)
                return cnt_v + plsc.all_reduce_population_count(m)

            cnt_v = lax.fori_loop(0, C // L, scan_body,
                                  jnp.zeros((L,), jnp.int32))
            nm = cnt_v[0]
            ng = (nm + (G - 1)) // G

            @pl.when(ng > 0)
            def _():
                fire_group(0, 0)

            @pl.loop(0, (ng + 1) // 2)
            def _(q):
                for gb in (0, 1):
                    g = 2 * q + gb

                    @pl.when(g < ng)
                    def _():
                        wait_group(gb)

                        @pl.when(g + 1 < ng)
                        def _():
                            fire_group(g + 1, 1 - gb)

                        process_group(g, gb)

    pltpu.sync_copy(dif_hbm.at[pl.ds(node_base, R)], difbuf)

    @pl.loop(0, R)
    def _(r):
        for f in range(NF):
            sl = pl.ds(f * L, L)
            a = acc[r, sl]
            acc[r, sl] = jnp.where(a == neg_inf, jnp.zeros((L,), jnp.float32),
                                   a + difbuf[r, sl])

    pltpu.sync_copy(acc.at[pl.ds(0, R)], out_hbm.at[pl.ds(node_base, R)])


def _sc_segment_max(h_pad, dif_pad, src, dst):
    mesh = plsc.VectorSubcoreMesh(core_axis_name="c", subcore_axis_name="s")
    return pl.kernel(
        _sc_body,
        out_type=jax.ShapeDtypeStruct((N_PAD, D), jnp.float32),
        mesh=mesh,
        compiler_params=pltpu.CompilerParams(needs_layout_passes=False),
        scratch_types=[
            pltpu.VMEM((R_ACC, D), jnp.float32),
            pltpu.VMEM((2, C), jnp.int32),
            pltpu.VMEM((2, C), jnp.int32),
            pltpu.VMEM((C,), jnp.int32),
            pltpu.VMEM((2, G), jnp.int32),
            pltpu.VMEM((2, G, D), jnp.float32),
            pltpu.VMEM((R, D), jnp.float32),
            pltpu.SemaphoreType.DMA,
            pltpu.SemaphoreType.DMA,
            pltpu.SemaphoreType.DMA,
            pltpu.SemaphoreType.DMA,
        ],
    )(h_pad, dif_pad, src, dst)


@jax.jit
def kernel(feat, edge_index, theta_w, theta_b, phi_w, phi_b):
    x_pad = jnp.zeros((M_PAD, D), jnp.float32).at[:N_NODES].set(feat)
    wt = theta_w.T
    wd = (phi_w - theta_w).T
    bt = jnp.tile(theta_b[None, :], (8, 1))
    bd = jnp.tile((phi_b - theta_b)[None, :], (8, 1))
    h_pad, dif_pad = _tc_matmuls(x_pad, wt, wd, bt, bd)

    src = edge_index[0].astype(jnp.int32)
    dst = edge_index[1].astype(jnp.int32)
    out_pad = _sc_segment_max(h_pad, dif_pad, src, dst)
    return out_pad[:N_NODES]


# speedup vs baseline: 2.7394x; 1.1411x over previous
import functools
import jax
import jax.numpy as jnp
from jax import lax
from jax.experimental import pallas as pl
from jax.experimental.pallas import tpu as pltpu
from jax.experimental.pallas import tpu_sc as plsc

N_NODES = 10000
N_EDGES = 320000
D = 128
L = 16
NF = D // L

NC, NS = 2, 16
NW = NC * NS
R = 320
N_PAD = NW * R
R_ACC = R + 1
DUMP = R
SENT = DUMP << 14

C = 6400
NCHUNK = N_EDGES // C
G = 32
GV = G // L

M_PAD = 10240
TC_BLK = 512


def _tc_body(x_ref, wt_ref, wd_ref, bt_ref, bd_ref, h_ref, d_ref):
    x = x_ref[...]
    h_ref[...] = jnp.dot(x, wt_ref[...], preferred_element_type=jnp.float32) + bt_ref[0:1, :]
    d_ref[...] = jnp.dot(x, wd_ref[...], preferred_element_type=jnp.float32) + bd_ref[0:1, :]


def _tc_matmuls(x_pad, wt, wd, bt, bd):
    grid = (M_PAD // TC_BLK,)
    return pl.pallas_call(
        _tc_body,
        grid=grid,
        in_specs=[
            pl.BlockSpec((TC_BLK, D), lambda i: (i, 0)),
            pl.BlockSpec((D, D), lambda i: (0, 0)),
            pl.BlockSpec((D, D), lambda i: (0, 0)),
            pl.BlockSpec((8, D), lambda i: (0, 0)),
            pl.BlockSpec((8, D), lambda i: (0, 0)),
        ],
        out_specs=[
            pl.BlockSpec((TC_BLK, D), lambda i: (i, 0)),
            pl.BlockSpec((TC_BLK, D), lambda i: (i, 0)),
        ],
        out_shape=[
            jax.ShapeDtypeStruct((M_PAD, D), jnp.float32),
            jax.ShapeDtypeStruct((M_PAD, D), jnp.float32),
        ],
    )(x_pad, wt, wd, bt, bd)


def _sc_body(h_hbm, dif_hbm, src_hbm, dst_hbm, out_hbm,
             acc, srcbuf, dstbuf, mkey, gidx, rows, difbuf,
             esem0, esem1, gsem0, gsem1):
    wid = lax.axis_index("s") * NC + lax.axis_index("c")
    node_base = wid * R

    neg_inf = jnp.full((L,), -jnp.inf, jnp.float32)
    esems = (esem0, esem1)
    gsems = (gsem0, gsem1)

    @pl.loop(0, R_ACC)
    def _(r):
        for f in range(NF):
            acc[r, pl.ds(f * L, L)] = neg_inf

    @pl.loop(0, C // L)
    def _(i):
        mkey[pl.ds(i * L, L)] = jnp.full((L,), SENT, jnp.int32)

    def fire_chunk(c, b):
        eb = c * C
        pltpu.async_copy(dst_hbm.at[pl.ds(eb, C)], dstbuf.at[b], esems[b])
        pltpu.async_copy(src_hbm.at[pl.ds(eb, C)], srcbuf.at[b], esems[b])

    def wait_chunk(b):
        pltpu.make_async_copy(dst_hbm.at[pl.ds(0, C)], dstbuf.at[b], esems[b]).wait()
        pltpu.make_async_copy(src_hbm.at[pl.ds(0, C)], srcbuf.at[b], esems[b]).wait()

    def fire_group(g, b):
        for k in range(GV):
            kv = mkey[pl.ds(g * G + k * L, L)]
            gidx[b, pl.ds(k * L, L)] = kv & 0x3FFF
        pltpu.async_copy(h_hbm.at[gidx.at[b]], rows.at[b], gsems[b])

    def wait_group(b):
        pltpu.make_async_copy(h_hbm.at[gidx.at[b]], rows.at[b], gsems[b]).wait()

    def process_group(g, b):
        for k in range(GV):
            kv = mkey[pl.ds(g * G + k * L, L)]
            ldv = jax.lax.shift_right_logical(kv, 14)
            for j in range(L):
                ld = ldv[j]
                jj = k * L + j
                for f in range(NF):
                    sl = pl.ds(f * L, L)
                    acc[ld, sl] = jnp.maximum(acc[ld, sl], rows[b, jj, sl])

    fire_chunk(0, 0)

    @pl.loop(0, NCHUNK // 2)
    def _(p):
        for b in (0, 1):
            c = 2 * p + b
            wait_chunk(b)

            @pl.when(c + 1 < NCHUNK)
            def _():
                fire_chunk(c + 1, 1 - b)

            def scan_body(i, cnt_v):
                for u in range(4):
                    off = (i * 4 + u) * L
                    d = dstbuf[b, pl.ds(off, L)]
                    s = srcbuf[b, pl.ds(off, L)]
                    ld = d - node_base
                    m = (ld >= 0) & (ld < R)
                    key = (ld << 14) | s
                    zero = jnp.zeros((L,), jnp.int32)
                    mi = jnp.where(m, jnp.ones((L,), jnp.int32), zero)
                    pc = plsc.cumsum(mi)
                    offs = cnt_v + pc - 1
                    plsc.store_scatter(mkey, [offs], key, mask=m)
                    cnt_v = cnt_v + plsc.all_reduce_population_count(m)
                return cnt_v

            cnt_v = lax.fori_loop(0, C // L // 4, scan_body,
                                  jnp.zeros((L,), jnp.int32))
            nm = cnt_v[0]
            ng = (nm + (G - 1)) >> 5

            @pl.when(ng > 0)
            def _():
                fire_group(0, 0)

            @pl.loop(0, (ng + 1) // 2)
            def _(q):
                for gb in (0, 1):
                    g = 2 * q + gb

                    @pl.when(g < ng)
                    def _():
                        wait_group(gb)

                        @pl.when(g + 1 < ng)
                        def _():
                            fire_group(g + 1, 1 - gb)

                        process_group(g, gb)

    pltpu.sync_copy(dif_hbm.at[pl.ds(node_base, R)], difbuf)

    @pl.loop(0, R)
    def _(r):
        for f in range(NF):
            sl = pl.ds(f * L, L)
            a = acc[r, sl]
            acc[r, sl] = jnp.where(a == neg_inf, jnp.zeros((L,), jnp.float32),
                                   a + difbuf[r, sl])

    pltpu.sync_copy(acc.at[pl.ds(0, R)], out_hbm.at[pl.ds(node_base, R)])


def _sc_segment_max(h_pad, dif_pad, src, dst):
    mesh = plsc.VectorSubcoreMesh(core_axis_name="c", subcore_axis_name="s")
    return pl.kernel(
        _sc_body,
        out_type=jax.ShapeDtypeStruct((N_PAD, D), jnp.float32),
        mesh=mesh,
        compiler_params=pltpu.CompilerParams(needs_layout_passes=False),
        scratch_types=[
            pltpu.VMEM((R_ACC, D), jnp.float32),
            pltpu.VMEM((2, C), jnp.int32),
            pltpu.VMEM((2, C), jnp.int32),
            pltpu.VMEM((C,), jnp.int32),
            pltpu.VMEM((2, G), jnp.int32),
            pltpu.VMEM((2, G, D), jnp.float32),
            pltpu.VMEM((R, D), jnp.float32),
            pltpu.SemaphoreType.DMA,
            pltpu.SemaphoreType.DMA,
            pltpu.SemaphoreType.DMA,
            pltpu.SemaphoreType.DMA,
        ],
    )(h_pad, dif_pad, src, dst)


@jax.jit
def kernel(feat, edge_index, theta_w, theta_b, phi_w, phi_b):
    x_pad = jnp.zeros((M_PAD, D), jnp.float32).at[:N_NODES].set(feat)
    wt = theta_w.T
    wd = (phi_w - theta_w).T
    bt = jnp.tile(theta_b[None, :], (8, 1))
    bd = jnp.tile((phi_b - theta_b)[None, :], (8, 1))
    h_pad, dif_pad = _tc_matmuls(x_pad, wt, wd, bt, bd)

    src = edge_index[0].astype(jnp.int32)
    dst = edge_index[1].astype(jnp.int32)
    out_pad = _sc_segment_max(h_pad, dif_pad, src, dst)
    return out_pad[:N_NODES]


# speedup vs baseline: 2.7487x; 1.0034x over previous
import functools
import jax
import jax.numpy as jnp
from jax import lax
from jax.experimental import pallas as pl
from jax.experimental.pallas import tpu as pltpu
from jax.experimental.pallas import tpu_sc as plsc

N_NODES = 10000
N_EDGES = 320000
D = 128
L = 16
NF = D // L

NC, NS = 2, 16
NW = NC * NS
R = 320
N_PAD = NW * R
R_ACC = R + 1
DUMP = R
SENT = DUMP << 14

C = 6400
NCHUNK = N_EDGES // C
G = 32
GV = G // L

M_PAD = 10240
TC_BLK = 512


def _tc_body(x_ref, wt_ref, wd_ref, bt_ref, bd_ref, h_ref, d_ref):
    x = x_ref[...]
    h_ref[...] = jnp.dot(x, wt_ref[...], preferred_element_type=jnp.float32) + bt_ref[0:1, :]
    d_ref[...] = jnp.dot(x, wd_ref[...], preferred_element_type=jnp.float32) + bd_ref[0:1, :]


def _tc_matmuls(x_pad, wt, wd, bt, bd):
    grid = (M_PAD // TC_BLK,)
    return pl.pallas_call(
        _tc_body,
        grid=grid,
        in_specs=[
            pl.BlockSpec((TC_BLK, D), lambda i: (i, 0)),
            pl.BlockSpec((D, D), lambda i: (0, 0)),
            pl.BlockSpec((D, D), lambda i: (0, 0)),
            pl.BlockSpec((8, D), lambda i: (0, 0)),
            pl.BlockSpec((8, D), lambda i: (0, 0)),
        ],
        out_specs=[
            pl.BlockSpec((TC_BLK, D), lambda i: (i, 0)),
            pl.BlockSpec((TC_BLK, D), lambda i: (i, 0)),
        ],
        out_shape=[
            jax.ShapeDtypeStruct((M_PAD, D), jnp.float32),
            jax.ShapeDtypeStruct((M_PAD, D), jnp.float32),
        ],
    )(x_pad, wt, wd, bt, bd)


def _sc_body(h_hbm, dif_hbm, src_hbm, dst_hbm, out_hbm,
             acc, srcbuf, dstbuf, mkey, gidx, rows, difbuf,
             esem0, esem1, gsem0, gsem1):
    wid = lax.axis_index("s") * NC + lax.axis_index("c")
    node_base = wid * R

    neg_inf = jnp.full((L,), -jnp.inf, jnp.float32)
    esems = (esem0, esem1)
    gsems = (gsem0, gsem1)

    @pl.loop(0, R_ACC)
    def _(r):
        for f in range(NF):
            acc[r, pl.ds(f * L, L)] = neg_inf

    @pl.loop(0, C // L)
    def _(i):
        mkey[pl.ds(i * L, L)] = jnp.full((L,), SENT, jnp.int32)

    def fire_chunk(c, b):
        eb = c * C
        pltpu.async_copy(dst_hbm.at[pl.ds(eb, C)], dstbuf.at[b], esems[b])
        pltpu.async_copy(src_hbm.at[pl.ds(eb, C)], srcbuf.at[b], esems[b])

    def wait_chunk(b):
        pltpu.make_async_copy(dst_hbm.at[pl.ds(0, C)], dstbuf.at[b], esems[b]).wait()
        pltpu.make_async_copy(src_hbm.at[pl.ds(0, C)], srcbuf.at[b], esems[b]).wait()

    def fire_group(g, b):
        for k in range(GV):
            kv = mkey[pl.ds(g * G + k * L, L)]
            gidx[b, pl.ds(k * L, L)] = kv & 0x3FFF
        pltpu.async_copy(h_hbm.at[gidx.at[b]], rows.at[b], gsems[b])

    def wait_group(b):
        pltpu.make_async_copy(h_hbm.at[gidx.at[b]], rows.at[b], gsems[b]).wait()

    def process_group(g, b):
        for k in range(GV):
            kv = mkey[pl.ds(g * G + k * L, L)]
            ldv = jax.lax.shift_right_logical(kv, 14)
            for j in range(L):
                ld = ldv[j]
                jj = k * L + j
                for f in range(NF):
                    sl = pl.ds(f * L, L)
                    acc[ld, sl] = jnp.maximum(acc[ld, sl], rows[b, jj, sl])

    fire_chunk(0, 0)

    @pl.loop(0, NCHUNK // 2)
    def _(p):
        for b in (0, 1):
            c = 2 * p + b
            wait_chunk(b)

            @pl.when(c + 1 < NCHUNK)
            def _():
                fire_chunk(c + 1, 1 - b)

            def scan_body(i, cnt_v):
                for u in range(4):
                    off = (i * 4 + u) * L
                    d = dstbuf[b, pl.ds(off, L)]
                    s = srcbuf[b, pl.ds(off, L)]
                    ld = d - node_base
                    m = (ld >= 0) & (ld < R)
                    key = (ld << 14) | s
                    zero = jnp.zeros((L,), jnp.int32)
                    mi = jnp.where(m, jnp.ones((L,), jnp.int32), zero)
                    pc = plsc.cumsum(mi)
                    offs = cnt_v + pc - 1
                    plsc.store_scatter(mkey, [offs], key, mask=m)
                    cnt_v = cnt_v + plsc.all_reduce_population_count(m)
                return cnt_v

            with jax.named_scope("edge_scan"):
                cnt_v = lax.fori_loop(0, C // L // 4, scan_body,
                                      jnp.zeros((L,), jnp.int32))
            nm = cnt_v[0]
            ng = (nm + (G - 1)) >> 5

            @pl.when(ng > 0)
            def _():
                fire_group(0, 0)

            with jax.named_scope("gather_acc"):
                @pl.loop(0, (ng + 1) // 2)
                def _(q):
                    for gb in (0, 1):
                        g = 2 * q + gb

                        @pl.when(g < ng)
                        def _():
                            wait_group(gb)

                            @pl.when(g + 1 < ng)
                            def _():
                                fire_group(g + 1, 1 - gb)

                            process_group(g, gb)

    pltpu.sync_copy(dif_hbm.at[pl.ds(node_base, R)], difbuf)

    @pl.loop(0, R)
    def _(r):
        for f in range(NF):
            sl = pl.ds(f * L, L)
            a = acc[r, sl]
            acc[r, sl] = jnp.where(a == neg_inf, jnp.zeros((L,), jnp.float32),
                                   a + difbuf[r, sl])

    pltpu.sync_copy(acc.at[pl.ds(0, R)], out_hbm.at[pl.ds(node_base, R)])


def _sc_segment_max(h_pad, dif_pad, src, dst):
    mesh = plsc.VectorSubcoreMesh(core_axis_name="c", subcore_axis_name="s")
    return pl.kernel(
        _sc_body,
        out_type=jax.ShapeDtypeStruct((N_PAD, D), jnp.float32),
        mesh=mesh,
        compiler_params=pltpu.CompilerParams(needs_layout_passes=False),
        scratch_types=[
            pltpu.VMEM((R_ACC, D), jnp.float32),
            pltpu.VMEM((2, C), jnp.int32),
            pltpu.VMEM((2, C), jnp.int32),
            pltpu.VMEM((C,), jnp.int32),
            pltpu.VMEM((2, G), jnp.int32),
            pltpu.VMEM((2, G, D), jnp.float32),
            pltpu.VMEM((R, D), jnp.float32),
            pltpu.SemaphoreType.DMA,
            pltpu.SemaphoreType.DMA,
            pltpu.SemaphoreType.DMA,
            pltpu.SemaphoreType.DMA,
        ],
    )(h_pad, dif_pad, src, dst)


@jax.jit
def kernel(feat, edge_index, theta_w, theta_b, phi_w, phi_b):
    x_pad = jnp.zeros((M_PAD, D), jnp.float32).at[:N_NODES].set(feat)
    wt = theta_w.T
    wd = (phi_w - theta_w).T
    bt = jnp.tile(theta_b[None, :], (8, 1))
    bd = jnp.tile((phi_b - theta_b)[None, :], (8, 1))
    h_pad, dif_pad = _tc_matmuls(x_pad, wt, wd, bt, bd)

    src = edge_index[0].astype(jnp.int32)
    dst = edge_index[1].astype(jnp.int32)
    out_pad = _sc_segment_max(h_pad, dif_pad, src, dst)
    return out_pad[:N_NODES]
